# baseline (speedup 1.0000x reference)
import jax
import jax.numpy as jnp
from jax import lax
from jax.experimental import pallas as pl
from jax.experimental.pallas import tpu as pltpu
from jax.experimental.pallas import tpu_sc as plsc

N_NODES = 10000
N_EDGES = 320000
IN_CH = 128
H1 = 64
H2 = 32
MLP1 = 16
NUM_GRAPHS = 64

NC = 2
NS = 16
NW = NC * NS
C = 128
NCHUNK = N_EDGES // C
KM = NCHUNK // NW
NX = NCHUNK - KM * NW
R = 10112
STRIPE = R // NS
NB = 6

_mesh = plsc.VectorSubcoreMesh(core_axis_name="c", subcore_axis_name="s")
_sc_params = pltpu.CompilerParams(use_tc_tiling_on_sc=False)


def _worker_id():
    return lax.axis_index("c") * NS + lax.axis_index("s")


def _deg_body(ei3, ones_hbm, zeros_hbm, out, idx_c, idx_ce, ones_v, acc,
              sem):
    cid = lax.axis_index("c")
    sid = lax.axis_index("s")
    wid = _worker_id()
    extra = wid >= NW - NX
    pltpu.sync_copy(ei3.at[1, pl.ds(wid * KM, KM)], idx_c)

    @pl.when(extra)
    def _():
        pltpu.sync_copy(ei3.at[1, pl.ds(KM * NW + wid - (NW - NX), 1)],
                        idx_ce)

    pltpu.sync_copy(ones_hbm, ones_v)
    pltpu.sync_copy(zeros_hbm.at[pl.ds(sid * STRIPE, STRIPE)],
                    acc.at[pl.ds(sid * STRIPE, STRIPE)])
    plsc.subcore_barrier()

    def body(j, carry):
        pltpu.async_copy(ones_v, acc.at[idx_c.at[j]], sem, add=True)
        return carry

    lax.fori_loop(0, KM, body, 0)

    @pl.when(extra)
    def _():
        pltpu.async_copy(ones_v, acc.at[idx_ce.at[0]], sem, add=True)

    def drain(j, carry):
        pltpu.make_async_copy(ones_v, acc.at[idx_c.at[0]], sem).wait()
        return carry

    lax.fori_loop(0, KM, drain, 0)

    @pl.when(extra)
    def _():
        pltpu.make_async_copy(ones_v, acc.at[idx_c.at[0]], sem).wait()

    plsc.subcore_barrier()
    pltpu.sync_copy(acc.at[pl.ds(sid * STRIPE, STRIPE)],
                    out.at[pl.ds(sid * STRIPE, STRIPE), pl.ds(cid * 16, 16)])


_deg_kernel = pl.kernel(
    _deg_body,
    out_type=jax.ShapeDtypeStruct((R, 128), jnp.float32),
    mesh=_mesh,
    scratch_types=[
        pltpu.VMEM((KM, C), jnp.int32),
        pltpu.VMEM((1, C), jnp.int32),
        pltpu.VMEM((C, 16), jnp.float32),
        pltpu.VMEM_SHARED((R, 16), jnp.float32),
        pltpu.SemaphoreType.DMA,
    ],
    compiler_params=_sc_params,
)


def _agg_body(ei3, xs, zeros_hbm, out, idx_r, idx_c, idx_re, idx_ce,
              bufs, acc, sem_g, sem_s, *, d):
    cid = lax.axis_index("c")
    sid = lax.axis_index("s")
    wid = _worker_id()
    extra = wid >= NW - NX
    pltpu.sync_copy(ei3.at[0, pl.ds(wid * KM, KM)], idx_r)
    pltpu.sync_copy(ei3.at[1, pl.ds(wid * KM, KM)], idx_c)

    @pl.when(extra)
    def _():
        xo = KM * NW + wid - (NW - NX)
        pltpu.sync_copy(ei3.at[0, pl.ds(xo, 1)], idx_re)
        pltpu.sync_copy(ei3.at[1, pl.ds(xo, 1)], idx_ce)

    G = KM // NB

    def wait_gather(b):
        pltpu.make_async_copy(xs.at[idx_r.at[0]], bufs.at[b],
                              sem_g.at[b]).wait()

    def wait_scatter(b):
        pltpu.make_async_copy(bufs.at[b], acc.at[idx_c.at[0]],
                              sem_s.at[b]).wait()

    for b in range(NB):
        pltpu.async_copy(xs.at[idx_r.at[b]], bufs.at[b], sem_g.at[b])
    pltpu.sync_copy(zeros_hbm.at[pl.ds(sid * STRIPE, STRIPE)],
                    acc.at[pl.ds(sid * STRIPE, STRIPE)])
    plsc.subcore_barrier()

    def body(i, carry):
        for b in range(NB):
            wait_gather(b)
            pltpu.async_copy(bufs.at[b], acc.at[idx_c.at[i * NB + b]],
                             sem_s.at[b], add=True)

        @pl.when(i + 1 < G)
        def _():
            for b in range(NB):
                wait_scatter(b)
                pltpu.async_copy(xs.at[idx_r.at[(i + 1) * NB + b]],
                                 bufs.at[b], sem_g.at[b])

        return carry

    lax.fori_loop(0, G, body, 0)
    for b in range(NB):
        wait_scatter(b)

    @pl.when(extra)
    def _():
        pltpu.async_copy(xs.at[idx_re.at[0]], bufs.at[0], sem_g.at[0]).wait()
        pltpu.sync_copy(bufs.at[0], acc.at[idx_ce.at[0]], add=True)

    plsc.subcore_barrier()
    pltpu.sync_copy(acc.at[pl.ds(sid * STRIPE, STRIPE)],
                    out.at[pl.ds(sid * STRIPE, STRIPE), pl.ds(cid * d, d)])


def _make_agg(d):
    import functools
    return pl.kernel(
        functools.partial(_agg_body, d=d),
        out_type=jax.ShapeDtypeStruct((R, 128), jnp.float32),
        mesh=_mesh,
        scratch_types=[
            pltpu.VMEM((KM, C), jnp.int32),
            pltpu.VMEM((KM, C), jnp.int32),
            pltpu.VMEM((1, C), jnp.int32),
            pltpu.VMEM((1, C), jnp.int32),
            pltpu.VMEM((NB, C, d), jnp.float32),
            pltpu.VMEM_SHARED((R, d), jnp.float32),
            pltpu.SemaphoreType.DMA((NB,)),
            pltpu.SemaphoreType.DMA((NB,)),
        ],
        compiler_params=_sc_params,
    )


_agg64 = _make_agg(H1)
_agg32 = _make_agg(H2)


def _leaky(v):
    return jnp.where(v >= 0, v, 0.01 * v)


def _bn(h, g, be):
    mu = jnp.mean(h, axis=0, keepdims=True)
    var = jnp.mean((h - mu) * (h - mu), axis=0, keepdims=True)
    return (h - mu) * lax.rsqrt(var + 1e-5) * g + be


def _tc_a_body(degp_ref, x_ref, w1_ref, dinv_ref, xs1_ref):
    deg = degp_ref[:, 0:1] + degp_ref[:, 16:17] + 1.0
    dinv = lax.rsqrt(deg)
    dinv_ref[...] = dinv
    xs1_ref[...] = jnp.dot(x_ref[...] * dinv, w1_ref[...],
                           preferred_element_type=jnp.float32)


_BA = 1000
_tc_a_call = pl.pallas_call(
    _tc_a_body,
    grid=(N_NODES // _BA,),
    in_specs=[
        pl.BlockSpec((_BA, 128), lambda i: (i, 0)),
        pl.BlockSpec((_BA, IN_CH), lambda i: (i, 0)),
        pl.BlockSpec((IN_CH, H1), lambda i: (0, 0)),
    ],
    out_specs=[
        pl.BlockSpec((_BA, 1), lambda i: (i, 0)),
        pl.BlockSpec((_BA, H1), lambda i: (i, 0)),
    ],
    out_shape=[jax.ShapeDtypeStruct((N_NODES, 1), jnp.float32),
               jax.ShapeDtypeStruct((N_NODES, H1), jnp.float32)],
)


def _tc_b_body(sp_ref, xs1_ref, dinv_ref, b1_ref, g1_ref, be1_ref, w2_ref,
               xs2_ref):
    s = sp_ref[:N_NODES, 0:H1] + sp_ref[:N_NODES, H1:2 * H1]
    dv = dinv_ref[...]
    h = dv * (s + xs1_ref[...]) + b1_ref[...]
    h = _bn(_leaky(h), g1_ref[...], be1_ref[...])
    xs2_ref[...] = jnp.dot(h * dv, w2_ref[...],
                           preferred_element_type=jnp.float32)


def _tc_c_body(sp_ref, xs2_ref, dinv_ref, b2_ref, g2_ref, be2_ref, batch_ref,
               w3_ref, b3_ref, w4_ref, b4_ref, out_ref):
    s = sp_ref[:N_NODES, 0:H2] + sp_ref[:N_NODES, H2:2 * H2]
    dv = dinv_ref[...]
    h = dv * (s + xs2_ref[...]) + b2_ref[...]
    h = _bn(_leaky(h), g2_ref[...], be2_ref[...])
    seg = lax.broadcasted_iota(jnp.int32, (N_NODES, NUM_GRAPHS), 1)
    onehot = (seg == batch_ref[...]).astype(jnp.float32)
    pooled = lax.dot_general(onehot, h, (((0,), (0,)), ((), ())),
                             preferred_element_type=jnp.float32)
    z = _leaky(jnp.dot(pooled, w3_ref[...],
                       preferred_element_type=jnp.float32) + b3_ref[...])
    out_ref[...] = jnp.dot(z, w4_ref[...],
                           preferred_element_type=jnp.float32) + b4_ref[...]


def _tc_call(body, out_shapes):
    return pl.pallas_call(body, out_shape=out_shapes)


def kernel(x, edge_index, batch, W1, b1, g1, be1, W2, b2, g2, be2, W3, b3,
           W4, b4):
    ei3 = edge_index.reshape(2, NCHUNK, C)
    ones16 = jnp.ones((C, 16), jnp.float32)
    zeros16 = jnp.zeros((R, 16), jnp.float32)
    zeros64 = jnp.zeros((R, H1), jnp.float32)
    zeros32 = jnp.zeros((R, H2), jnp.float32)

    degp = _deg_kernel(ei3, ones16, zeros16)

    dinv, xs1 = _tc_a_call(degp, x, W1)

    sp1 = _agg64(ei3, xs1, zeros64)

    xs2 = _tc_call(
        _tc_b_body,
        jax.ShapeDtypeStruct((N_NODES, H2), jnp.float32),
    )(sp1, xs1, dinv, b1.reshape(1, H1), g1.reshape(1, H1),
      be1.reshape(1, H1), W2)

    sp2 = _agg32(ei3, xs2, zeros32)

    out = _tc_call(
        _tc_c_body,
        jax.ShapeDtypeStruct((NUM_GRAPHS, 1), jnp.float32),
    )(sp2, xs2, dinv, b2.reshape(1, H2), g2.reshape(1, H2),
      be2.reshape(1, H2), batch.reshape(N_NODES, 1), W3,
      b3.reshape(1, MLP1), W4, b4.reshape(1, 1))

    return out


# speedup vs baseline: 1.0205x; 1.0205x over previous
import jax
import jax.numpy as jnp
from jax import lax
from jax.experimental import pallas as pl
from jax.experimental.pallas import tpu as pltpu
from jax.experimental.pallas import tpu_sc as plsc

N_NODES = 10000
N_EDGES = 320000
IN_CH = 128
H1 = 64
H2 = 32
MLP1 = 16
NUM_GRAPHS = 64

NC = 2
NS = 16
NW = NC * NS
C = 128
NCHUNK = N_EDGES // C
KM = NCHUNK // NW
NX = NCHUNK - KM * NW
R = 10112
STRIPE = R // NS
NB = 6

_mesh = plsc.VectorSubcoreMesh(core_axis_name="c", subcore_axis_name="s")
_sc_params = pltpu.CompilerParams(use_tc_tiling_on_sc=False)


def _worker_id():
    return lax.axis_index("c") * NS + lax.axis_index("s")


def _deg_body(ei3, ones_hbm, zeros_hbm, out, idx_c, idx_ce, ones_v, acc,
              sem):
    cid = lax.axis_index("c")
    sid = lax.axis_index("s")
    wid = _worker_id()
    extra = wid >= NW - NX
    pltpu.sync_copy(ei3.at[1, pl.ds(wid * KM, KM)], idx_c)

    @pl.when(extra)
    def _():
        pltpu.sync_copy(ei3.at[1, pl.ds(KM * NW + wid - (NW - NX), 1)],
                        idx_ce)

    pltpu.sync_copy(ones_hbm, ones_v)
    pltpu.sync_copy(zeros_hbm.at[pl.ds(sid * STRIPE, STRIPE)],
                    acc.at[pl.ds(sid * STRIPE, STRIPE)])
    plsc.subcore_barrier()

    def body(j, carry):
        pltpu.async_copy(ones_v, acc.at[idx_c.at[j]], sem, add=True)
        return carry

    lax.fori_loop(0, KM, body, 0)

    @pl.when(extra)
    def _():
        pltpu.async_copy(ones_v, acc.at[idx_ce.at[0]], sem, add=True)

    def drain(j, carry):
        pltpu.make_async_copy(ones_v, acc.at[idx_c.at[0]], sem).wait()
        return carry

    lax.fori_loop(0, KM, drain, 0)

    @pl.when(extra)
    def _():
        pltpu.make_async_copy(ones_v, acc.at[idx_c.at[0]], sem).wait()

    plsc.subcore_barrier()
    pltpu.sync_copy(acc.at[pl.ds(sid * STRIPE, STRIPE)],
                    out.at[pl.ds(sid * STRIPE, STRIPE), pl.ds(cid * 16, 16)])


_deg_kernel = pl.kernel(
    _deg_body,
    out_type=jax.ShapeDtypeStruct((R, 128), jnp.float32),
    mesh=_mesh,
    scratch_types=[
        pltpu.VMEM((KM, C), jnp.int32),
        pltpu.VMEM((1, C), jnp.int32),
        pltpu.VMEM((C, 16), jnp.float32),
        pltpu.VMEM_SHARED((R, 16), jnp.float32),
        pltpu.SemaphoreType.DMA,
    ],
    compiler_params=_sc_params,
)


def _agg_body(ei3, xs, zeros_hbm, out, idx_r, idx_c, idx_re, idx_ce,
              bufs, acc, sem_g, sem_s, *, d):
    cid = lax.axis_index("c")
    sid = lax.axis_index("s")
    wid = _worker_id()
    extra = wid >= NW - NX
    pltpu.sync_copy(ei3.at[0, pl.ds(wid * KM, KM)], idx_r)
    pltpu.sync_copy(ei3.at[1, pl.ds(wid * KM, KM)], idx_c)

    @pl.when(extra)
    def _():
        xo = KM * NW + wid - (NW - NX)
        pltpu.sync_copy(ei3.at[0, pl.ds(xo, 1)], idx_re)
        pltpu.sync_copy(ei3.at[1, pl.ds(xo, 1)], idx_ce)

    G = KM // NB

    def wait_gather(b):
        pltpu.make_async_copy(xs.at[idx_r.at[0]], bufs.at[b],
                              sem_g.at[b]).wait()

    def wait_scatter(b):
        pltpu.make_async_copy(bufs.at[b], acc.at[idx_c.at[0]],
                              sem_s.at[b]).wait()

    for b in range(NB):
        pltpu.async_copy(xs.at[idx_r.at[b]], bufs.at[b], sem_g.at[b])
    pltpu.sync_copy(zeros_hbm.at[pl.ds(sid * STRIPE, STRIPE)],
                    acc.at[pl.ds(sid * STRIPE, STRIPE)])
    plsc.subcore_barrier()

    def body(i, carry):
        for b in range(NB):
            wait_gather(b)
            pltpu.async_copy(bufs.at[b], acc.at[idx_c.at[i * NB + b]],
                             sem_s.at[b], add=True)

        @pl.when(i + 1 < G)
        def _():
            for b in range(NB):
                wait_scatter(b)
                pltpu.async_copy(xs.at[idx_r.at[(i + 1) * NB + b]],
                                 bufs.at[b], sem_g.at[b])

        return carry

    lax.fori_loop(0, G, body, 0)
    for b in range(NB):
        wait_scatter(b)

    @pl.when(extra)
    def _():
        pltpu.async_copy(xs.at[idx_re.at[0]], bufs.at[0], sem_g.at[0]).wait()
        pltpu.sync_copy(bufs.at[0], acc.at[idx_ce.at[0]], add=True)

    plsc.subcore_barrier()
    pltpu.sync_copy(acc.at[pl.ds(sid * STRIPE, STRIPE)],
                    out.at[pl.ds(sid * STRIPE, STRIPE), pl.ds(cid * d, d)])


def _make_agg(d):
    import functools
    return pl.kernel(
        functools.partial(_agg_body, d=d),
        out_type=jax.ShapeDtypeStruct((R, 128), jnp.float32),
        mesh=_mesh,
        scratch_types=[
            pltpu.VMEM((KM, C), jnp.int32),
            pltpu.VMEM((KM, C), jnp.int32),
            pltpu.VMEM((1, C), jnp.int32),
            pltpu.VMEM((1, C), jnp.int32),
            pltpu.VMEM((NB, C, d), jnp.float32),
            pltpu.VMEM_SHARED((R, d), jnp.float32),
            pltpu.SemaphoreType.DMA((NB,)),
            pltpu.SemaphoreType.DMA((NB,)),
        ],
        compiler_params=_sc_params,
    )


_agg64 = _make_agg(H1)
_agg32 = _make_agg(H2)


def _leaky(v):
    return jnp.where(v >= 0, v, 0.01 * v)


def _bn(h, g, be):
    mu = jnp.mean(h, axis=0, keepdims=True)
    var = jnp.mean((h - mu) * (h - mu), axis=0, keepdims=True)
    return (h - mu) * lax.rsqrt(var + 1e-5) * g + be


def _tc_a_body(degp_ref, x_ref, w1_ref, dinv_ref, xs1_ref):
    deg = degp_ref[:N_NODES, 0:1] + degp_ref[:N_NODES, 16:17] + 1.0
    dinv = lax.rsqrt(deg)
    dinv_ref[...] = dinv
    xs1_ref[...] = jnp.dot(x_ref[...] * dinv, w1_ref[...],
                           preferred_element_type=jnp.float32)


_tc_a_call = pl.pallas_call(
    _tc_a_body,
    out_shape=[jax.ShapeDtypeStruct((N_NODES, 1), jnp.float32),
               jax.ShapeDtypeStruct((N_NODES, H1), jnp.float32)],
)


def _tc_b_body(sp_ref, xs1_ref, dinv_ref, b1_ref, g1_ref, be1_ref, w2_ref,
               xs2_ref):
    s = sp_ref[:N_NODES, 0:H1] + sp_ref[:N_NODES, H1:2 * H1]
    dv = dinv_ref[...]
    h = dv * (s + xs1_ref[...]) + b1_ref[...]
    h = _bn(_leaky(h), g1_ref[...], be1_ref[...])
    xs2_ref[...] = jnp.dot(h * dv, w2_ref[...],
                           preferred_element_type=jnp.float32)


def _tc_c_body(sp_ref, xs2_ref, dinv_ref, b2_ref, g2_ref, be2_ref, batch_ref,
               w3_ref, b3_ref, w4_ref, b4_ref, out_ref):
    s = sp_ref[:N_NODES, 0:H2] + sp_ref[:N_NODES, H2:2 * H2]
    dv = dinv_ref[...]
    h = dv * (s + xs2_ref[...]) + b2_ref[...]
    h = _bn(_leaky(h), g2_ref[...], be2_ref[...])
    seg = lax.broadcasted_iota(jnp.int32, (N_NODES, NUM_GRAPHS), 1)
    onehot = (seg == batch_ref[...]).astype(jnp.float32)
    pooled = lax.dot_general(onehot, h, (((0,), (0,)), ((), ())),
                             preferred_element_type=jnp.float32)
    z = _leaky(jnp.dot(pooled, w3_ref[...],
                       preferred_element_type=jnp.float32) + b3_ref[...])
    out_ref[...] = jnp.dot(z, w4_ref[...],
                           preferred_element_type=jnp.float32) + b4_ref[...]


def _tc_call(body, out_shapes):
    return pl.pallas_call(body, out_shape=out_shapes)


def kernel(x, edge_index, batch, W1, b1, g1, be1, W2, b2, g2, be2, W3, b3,
           W4, b4):
    ei3 = edge_index.reshape(2, NCHUNK, C)
    ones16 = jnp.ones((C, 16), jnp.float32)
    zeros16 = jnp.zeros((R, 16), jnp.float32)
    zeros64 = jnp.zeros((R, H1), jnp.float32)
    zeros32 = jnp.zeros((R, H2), jnp.float32)

    degp = _deg_kernel(ei3, ones16, zeros16)

    dinv, xs1 = _tc_a_call(degp, x, W1)

    sp1 = _agg64(ei3, xs1, zeros64)

    xs2 = _tc_call(
        _tc_b_body,
        jax.ShapeDtypeStruct((N_NODES, H2), jnp.float32),
    )(sp1, xs1, dinv, b1.reshape(1, H1), g1.reshape(1, H1),
      be1.reshape(1, H1), W2)

    sp2 = _agg32(ei3, xs2, zeros32)

    out = _tc_call(
        _tc_c_body,
        jax.ShapeDtypeStruct((NUM_GRAPHS, 1), jnp.float32),
    )(sp2, xs2, dinv, b2.reshape(1, H2), g2.reshape(1, H2),
      be2.reshape(1, H2), batch.reshape(N_NODES, 1), W3,
      b3.reshape(1, MLP1), W4, b4.reshape(1, 1))

    return out
